# baseline (device time: 46806 ns/iter reference)
import jax
import jax.numpy as jnp
from jax import lax
from jax.experimental import pallas as pl
from jax.experimental.pallas import tpu as pltpu

N_DEV = 4


def kernel(x, w_mat):
    m_per, k = x.shape
    _, n_per = w_mat.shape

    def body(x_ref, w_ref, out_ref, comm_ref, send_sems, recv_sems):
        my_pos = lax.axis_index("i")
        left = (my_pos - 1) % N_DEV
        right = (my_pos + 1) % N_DEV

        barrier_sem = pltpu.get_barrier_semaphore()
        for nbr in (left, right):
            pl.semaphore_signal(
                barrier_sem, inc=1,
                device_id=(nbr,), device_id_type=pl.DeviceIdType.MESH,
            )
        pl.semaphore_wait(barrier_sem, 2)

        comm_ref[0, :, :] = x_ref[:, :]

        for h in range(N_DEV - 1):
            rdma = pltpu.make_async_remote_copy(
                src_ref=comm_ref.at[h],
                dst_ref=comm_ref.at[h + 1],
                send_sem=send_sems.at[h],
                recv_sem=recv_sems.at[h],
                device_id=(right,),
                device_id_type=pl.DeviceIdType.MESH,
            )
            rdma.start()
            if h == 0:
                out_ref[pl.ds(my_pos * m_per, m_per), :] = jnp.dot(
                    x_ref[:, :], w_ref[:, :],
                    preferred_element_type=jnp.float32,
                )
            rdma.wait()
            origin = (my_pos - h - 1) % N_DEV
            out_ref[pl.ds(origin * m_per, m_per), :] = jnp.dot(
                comm_ref[h + 1, :, :], w_ref[:, :],
                preferred_element_type=jnp.float32,
            )

    return pl.pallas_call(
        body,
        out_shape=jax.ShapeDtypeStruct((N_DEV * m_per, n_per), jnp.float32),
        in_specs=[
            pl.BlockSpec(memory_space=pltpu.VMEM),
            pl.BlockSpec(memory_space=pltpu.VMEM),
        ],
        out_specs=pl.BlockSpec(memory_space=pltpu.VMEM),
        scratch_shapes=[
            pltpu.VMEM((N_DEV, m_per, k), jnp.float32),
            pltpu.SemaphoreType.DMA((N_DEV - 1,)),
            pltpu.SemaphoreType.DMA((N_DEV - 1,)),
        ],
        compiler_params=pltpu.CompilerParams(collective_id=0),
    )(x, w_mat)


# device time: 29103 ns/iter; 1.6083x vs baseline; 1.6083x over previous
import jax
import jax.numpy as jnp
from jax import lax
from jax.experimental import pallas as pl
from jax.experimental.pallas import tpu as pltpu

N_DEV = 4


def kernel(x, w_mat):
    m_per, k = x.shape
    _, n_per = w_mat.shape
    half = m_per // 2

    def body(x_ref, w_ref, out_ref, cw_ref, ccw_ref,
             cw_send, cw_recv, ccw_send, ccw_recv):
        my_pos = lax.axis_index("i")
        left = (my_pos - 1) % N_DEV
        right = (my_pos + 1) % N_DEV

        barrier_sem = pltpu.get_barrier_semaphore()
        for nbr in (left, right):
            pl.semaphore_signal(
                barrier_sem, inc=1,
                device_id=(nbr,), device_id_type=pl.DeviceIdType.MESH,
            )
        pl.semaphore_wait(barrier_sem, 2)

        cw_ref[0, :, :] = x_ref[:half, :]
        ccw_ref[0, :, :] = x_ref[half:, :]

        def make_hop(h):
            cw = pltpu.make_async_remote_copy(
                src_ref=cw_ref.at[h],
                dst_ref=cw_ref.at[h + 1],
                send_sem=cw_send.at[h],
                recv_sem=cw_recv.at[h],
                device_id=(right,),
                device_id_type=pl.DeviceIdType.MESH,
            )
            ccw = pltpu.make_async_remote_copy(
                src_ref=ccw_ref.at[h],
                dst_ref=ccw_ref.at[h + 1],
                send_sem=ccw_send.at[h],
                recv_sem=ccw_recv.at[h],
                device_id=(left,),
                device_id_type=pl.DeviceIdType.MESH,
            )
            return cw, ccw

        def compute_halves(h):
            o_cw = (my_pos - h) % N_DEV
            out_ref[pl.ds(o_cw * m_per, half), :] = jnp.dot(
                cw_ref[h, :, :], w_ref[:, :],
                preferred_element_type=jnp.float32,
            )
            o_ccw = (my_pos + h) % N_DEV
            out_ref[pl.ds(o_ccw * m_per + half, half), :] = jnp.dot(
                ccw_ref[h, :, :], w_ref[:, :],
                preferred_element_type=jnp.float32,
            )

        cw0, ccw0 = make_hop(0)
        cw0.start()
        ccw0.start()
        out_ref[pl.ds(my_pos * m_per, m_per), :] = jnp.dot(
            x_ref[:, :], w_ref[:, :], preferred_element_type=jnp.float32,
        )
        cw0.wait()
        ccw0.wait()

        for h in range(1, N_DEV - 1):
            cwh, ccwh = make_hop(h)
            cwh.start()
            ccwh.start()
            compute_halves(h)
            cwh.wait()
            ccwh.wait()

        compute_halves(N_DEV - 1)

    return pl.pallas_call(
        body,
        out_shape=jax.ShapeDtypeStruct((N_DEV * m_per, n_per), jnp.float32),
        in_specs=[
            pl.BlockSpec(memory_space=pltpu.VMEM),
            pl.BlockSpec(memory_space=pltpu.VMEM),
        ],
        out_specs=pl.BlockSpec(memory_space=pltpu.VMEM),
        scratch_shapes=[
            pltpu.VMEM((N_DEV, half, k), jnp.float32),
            pltpu.VMEM((N_DEV, half, k), jnp.float32),
            pltpu.SemaphoreType.DMA((N_DEV - 1,)),
            pltpu.SemaphoreType.DMA((N_DEV - 1,)),
            pltpu.SemaphoreType.DMA((N_DEV - 1,)),
            pltpu.SemaphoreType.DMA((N_DEV - 1,)),
        ],
        compiler_params=pltpu.CompilerParams(collective_id=0),
    )(x, w_mat)


# device time: 24607 ns/iter; 1.9021x vs baseline; 1.1827x over previous
import jax
import jax.numpy as jnp
from jax import lax
from jax.experimental import pallas as pl
from jax.experimental.pallas import tpu as pltpu

N_DEV = 4
N_SUB = 2


def kernel(x, w_mat):
    m_per, k = x.shape
    _, n_per = w_mat.shape
    half = m_per // 2
    sub = half // N_SUB

    def body(x_ref, w_ref, out_ref, cw_ref, ccw_ref,
             cw_send, cw_recv, ccw_send, ccw_recv):
        my_pos = lax.axis_index("i")
        left = (my_pos - 1) % N_DEV
        right = (my_pos + 1) % N_DEV

        barrier_sem = pltpu.get_barrier_semaphore()
        for nbr in (left, right):
            pl.semaphore_signal(
                barrier_sem, inc=1,
                device_id=(nbr,), device_id_type=pl.DeviceIdType.MESH,
            )
        pl.semaphore_wait(barrier_sem, 2)

        def make_hop(d, s, h):
            buf, send, recv, tgt, row0 = (
                (cw_ref, cw_send, cw_recv, right, s * sub)
                if d == 0
                else (ccw_ref, ccw_send, ccw_recv, left, half + s * sub)
            )
            src = x_ref.at[pl.ds(row0, sub), :] if h == 0 else buf.at[s, h - 1]
            return pltpu.make_async_remote_copy(
                src_ref=src,
                dst_ref=buf.at[s, h],
                send_sem=send.at[s, h],
                recv_sem=recv.at[s, h],
                device_id=(tgt,),
                device_id_type=pl.DeviceIdType.MESH,
            )

        def gemm(d, s, h):
            if d == 0:
                origin = (my_pos - h - 1) % N_DEV
                buf, row0 = cw_ref, s * sub
            else:
                origin = (my_pos + h + 1) % N_DEV
                buf, row0 = ccw_ref, half + s * sub
            out_ref[pl.ds(origin * m_per + row0, sub), :] = jnp.dot(
                buf[s, h, :, :], w_ref[:, :],
                preferred_element_type=jnp.float32,
            )

        hop0 = [make_hop(d, s, 0) for d in (0, 1) for s in range(N_SUB)]
        for r in hop0:
            r.start()
        out_ref[pl.ds(my_pos * m_per, m_per), :] = jnp.dot(
            x_ref[:, :], w_ref[:, :], preferred_element_type=jnp.float32,
        )

        prev = {(d, s): r for r, (d, s) in
                zip(hop0, [(d, s) for d in (0, 1) for s in range(N_SUB)])}
        for h in range(1, N_DEV - 1):
            for s in range(N_SUB):
                for d in (0, 1):
                    prev[(d, s)].wait()
                    nxt = make_hop(d, s, h)
                    nxt.start()
                    prev[(d, s)] = nxt
                for d in (0, 1):
                    gemm(d, s, h - 1)

        for s in range(N_SUB):
            for d in (0, 1):
                prev[(d, s)].wait()
            for d in (0, 1):
                gemm(d, s, N_DEV - 2)

    return pl.pallas_call(
        body,
        out_shape=jax.ShapeDtypeStruct((N_DEV * m_per, n_per), jnp.float32),
        in_specs=[
            pl.BlockSpec(memory_space=pltpu.VMEM),
            pl.BlockSpec(memory_space=pltpu.VMEM),
        ],
        out_specs=pl.BlockSpec(memory_space=pltpu.VMEM),
        scratch_shapes=[
            pltpu.VMEM((N_SUB, N_DEV - 1, sub, k), jnp.float32),
            pltpu.VMEM((N_SUB, N_DEV - 1, sub, k), jnp.float32),
            pltpu.SemaphoreType.DMA((N_SUB, N_DEV - 1)),
            pltpu.SemaphoreType.DMA((N_SUB, N_DEV - 1)),
            pltpu.SemaphoreType.DMA((N_SUB, N_DEV - 1)),
            pltpu.SemaphoreType.DMA((N_SUB, N_DEV - 1)),
        ],
        compiler_params=pltpu.CompilerParams(collective_id=0),
    )(x, w_mat)
